# baseline (speedup 1.0000x reference)
import functools
import jax
import jax.numpy as jnp
from jax import lax
from jax.experimental import pallas as pl
from jax.experimental.pallas import tpu as pltpu
from jax.experimental.pallas import tpu_sc as plsc

_E = 8
_T = 2048
_D = 2048
_FF = 2048
_P = 2 * _T
_BLK = 128
_NSLOTS = _P + _E * _BLK
_NBLK = _NSLOTS // _BLK
_CH = 16
_ROUTE_CHUNK = 256


def _route_kernel(x_ref, wg_ref, pos_ref, w_ref, bex_ref, oh_ref, rank_ref):
    logits = jnp.dot(x_ref[...], wg_ref[...], preferred_element_type=jnp.float32)
    cols = lax.broadcasted_iota(jnp.int32, logits.shape, 1)
    v1 = jnp.max(logits, axis=1, keepdims=True)
    a1 = jnp.argmax(logits, axis=1).astype(jnp.int32)
    masked = jnp.where(cols == a1[:, None], -jnp.inf, logits)
    v2 = jnp.max(masked, axis=1, keepdims=True)
    a2 = jnp.argmax(masked, axis=1).astype(jnp.int32)
    r = jnp.exp(v2 - v1)
    w1c = 1.0 / (1.0 + r)
    w2c = r * w1c

    e_pair = jnp.concatenate([a1, a2], axis=0)
    oh = (e_pair[:, None] == lax.broadcasted_iota(jnp.int32, (_P, _E), 1))
    oh_ref[...] = oh.astype(jnp.float32)

    n_ch = _P // _ROUTE_CHUNK
    ri = lax.broadcasted_iota(jnp.int32, (_ROUTE_CHUNK, _ROUTE_CHUNK), 0)
    ci = lax.broadcasted_iota(jnp.int32, (_ROUTE_CHUNK, _ROUTE_CHUNK), 1)
    tril = (ci <= ri).astype(jnp.float32)

    def body(c, carry):
        rows = oh_ref[pl.ds(c * _ROUTE_CHUNK, _ROUTE_CHUNK), :]
        part = jnp.dot(tril, rows, preferred_element_type=jnp.float32) + carry
        rank_ref[pl.ds(c * _ROUTE_CHUNK, _ROUTE_CHUNK), :] = part
        return carry + jnp.sum(rows, axis=0, keepdims=True)

    total = lax.fori_loop(0, n_ch, body, jnp.zeros((1, _E), jnp.float32))

    padded = jnp.floor((total + float(_BLK - 1)) / float(_BLK)) * float(_BLK)
    ei = lax.broadcasted_iota(jnp.int32, (_E, _E), 0)
    ej = lax.broadcasted_iota(jnp.int32, (_E, _E), 1)
    sut = (ei < ej).astype(jnp.float32)
    base = jnp.dot(padded, sut, preferred_element_type=jnp.float32)

    rank = rank_ref[...]
    posm = base + rank - 1.0
    pos_pair = jnp.sum(jnp.where(oh, posm, 0.0), axis=1)
    pos0 = pos_pair[: _T].astype(jnp.int32)
    pos1 = pos_pair[_T:].astype(jnp.int32)

    lane = lax.broadcasted_iota(jnp.int32, (_T, 128), 1)
    pos_ref[...] = jnp.where(lane == 0, pos0[:, None],
                             jnp.where(lane == 1, pos1[:, None], 0))
    w_ref[...] = jnp.where(lane == 0, w1c,
                           jnp.where(lane == 1, w2c, 0.0))

    blane = lax.broadcasted_iota(jnp.int32, (8, 128), 1)
    s = (blane * _BLK).astype(jnp.float32)
    be = jnp.zeros((8, 128), jnp.float32)
    for e in range(_E):
        lane8 = lax.broadcasted_iota(jnp.int32, (1, _E), 1)
        b_e = jnp.sum(jnp.where(lane8 == e, base, 0.0))
        p_e = jnp.sum(jnp.where(lane8 == e, padded, 0.0))
        be = jnp.where((s >= b_e) & (s < b_e + p_e), float(e), be)
    nact = jnp.sum(padded) / float(_BLK)
    be = jnp.where((blane == _NBLK) & (lax.broadcasted_iota(jnp.int32, (8, 128), 0) == 0),
                   nact, be)
    bex_ref[...] = be.astype(jnp.int32)


def _h_kernel(bex_ref, xd_ref, w1_ref, h_ref):
    @pl.when(pl.program_id(0) < bex_ref[_NBLK])
    def _():
        h = jnp.dot(xd_ref[...], w1_ref[0], preferred_element_type=jnp.float32)
        h_ref[...] = (h * lax.logistic(h)).astype(jnp.bfloat16)


def _y_kernel(bex_ref, h_ref, w2_ref, y_ref):
    @pl.when(pl.program_id(0) < bex_ref[_NBLK])
    def _():
        y_ref[...] = jnp.dot(h_ref[...].astype(jnp.float32), w2_ref[0],
                             preferred_element_type=jnp.float32)


def _combine_kernel(w_ref, y0_ref, y1_ref, out_ref):
    w = w_ref[...]
    lane = lax.broadcasted_iota(jnp.int32, w.shape, 1)
    w0 = jnp.sum(jnp.where(lane == 0, w, 0.0), axis=1, keepdims=True)
    w1 = jnp.sum(jnp.where(lane == 1, w, 0.0), axis=1, keepdims=True)
    out_ref[...] = w0 * y0_ref[...] + w1 * y1_ref[...]


@functools.lru_cache(maxsize=1)
def _make_sc_kernels():
    info = plsc.get_sparse_core_info()
    nw = info.num_cores * info.num_subcores
    per_w = _P // nw
    n_ch = per_w // _CH

    mesh = plsc.VectorSubcoreMesh(core_axis_name="c", subcore_axis_name="s")

    @functools.partial(
        pl.kernel, mesh=mesh,
        out_type=jax.ShapeDtypeStruct((_NSLOTS, _D), jnp.float32),
        scratch_types=[
            pltpu.VMEM((per_w,), jnp.int32),
            pltpu.VMEM((_CH, _D), jnp.float32),
            pltpu.VMEM((_CH, _D), jnp.float32),
            pltpu.SemaphoreType.DMA,
            pltpu.SemaphoreType.DMA,
        ],
    )
    def dispatch(x_hbm, pos_hbm, xd_hbm, pos_v, buf_a, buf_b, sem_a, sem_b):
        wid = lax.axis_index("s") * info.num_cores + lax.axis_index("c")
        p0 = wid * per_w
        row0 = lax.rem(p0, _T)
        pltpu.sync_copy(pos_hbm.at[pl.ds(p0, per_w)], pos_v)
        bufs = (buf_a, buf_b)
        sems = (sem_a, sem_b)
        descs = [None, None]
        for j in range(n_ch):
            b = j % 2
            if descs[b] is not None:
                descs[b].wait()
            pltpu.sync_copy(x_hbm.at[pl.ds(row0 + j * _CH, _CH)], bufs[b])
            idx = pos_v[pl.ds(j * _CH, _CH)]
            descs[b] = pltpu.async_copy(bufs[b], xd_hbm.at[idx], sems[b])
        for d in descs:
            if d is not None:
                d.wait()

    @functools.partial(
        pl.kernel, mesh=mesh,
        out_type=jax.ShapeDtypeStruct((_P, _D), jnp.float32),
        scratch_types=[
            pltpu.VMEM((per_w,), jnp.int32),
            pltpu.VMEM((_CH, _D), jnp.float32),
            pltpu.VMEM((_CH, _D), jnp.float32),
            pltpu.SemaphoreType.DMA,
            pltpu.SemaphoreType.DMA,
        ],
    )
    def gather(y_hbm, pos_hbm, yg_hbm, pos_v, buf_a, buf_b, sem_a, sem_b):
        wid = lax.axis_index("s") * info.num_cores + lax.axis_index("c")
        p0 = wid * per_w
        pltpu.sync_copy(pos_hbm.at[pl.ds(p0, per_w)], pos_v)
        bufs = (buf_a, buf_b)
        sems = (sem_a, sem_b)
        descs = [None, None]
        for j in range(n_ch):
            b = j % 2
            if descs[b] is not None:
                descs[b].wait()
                pltpu.sync_copy(bufs[b], yg_hbm.at[pl.ds(p0 + (j - 2) * _CH, _CH)])
            idx = pos_v[pl.ds(j * _CH, _CH)]
            descs[b] = pltpu.async_copy(y_hbm.at[idx], bufs[b], sems[b])
        for j in range(n_ch - 2, n_ch):
            b = j % 2
            descs[b].wait()
            pltpu.sync_copy(bufs[b], yg_hbm.at[pl.ds(p0 + j * _CH, _CH)])

    return dispatch, gather


@jax.jit
def kernel(input, Wg, W1, W2):
    _dispatch_sc, _gather_sc = _make_sc_kernels()
    pos128, w128, bex128 = pl.pallas_call(
        _route_kernel,
        out_shape=[
            jax.ShapeDtypeStruct((_T, 128), jnp.int32),
            jax.ShapeDtypeStruct((_T, 128), jnp.float32),
            jax.ShapeDtypeStruct((8, 128), jnp.int32),
        ],
        scratch_shapes=[
            pltpu.VMEM((_P, _E), jnp.float32),
            pltpu.VMEM((_P, _E), jnp.float32),
        ],
    )(input, Wg)

    pos_flat = jnp.concatenate([pos128[:, 0], pos128[:, 1]], axis=0)
    bex = bex128[0, : _NBLK + 8]

    xd = _dispatch_sc(input, pos_flat)

    grid_spec = pltpu.PrefetchScalarGridSpec(
        num_scalar_prefetch=1,
        grid=(_NBLK,),
        in_specs=[
            pl.BlockSpec((_BLK, _D), lambda b, bex_ref: (b, 0)),
            pl.BlockSpec((1, _D, _FF), lambda b, bex_ref: (bex_ref[b], 0, 0)),
        ],
        out_specs=pl.BlockSpec((_BLK, _FF), lambda b, bex_ref: (b, 0)),
    )
    h = pl.pallas_call(
        _h_kernel,
        grid_spec=grid_spec,
        out_shape=jax.ShapeDtypeStruct((_NSLOTS, _FF), jnp.bfloat16),
    )(bex, xd, W1)

    grid_spec2 = pltpu.PrefetchScalarGridSpec(
        num_scalar_prefetch=1,
        grid=(_NBLK,),
        in_specs=[
            pl.BlockSpec((_BLK, _FF), lambda b, bex_ref: (b, 0)),
            pl.BlockSpec((1, _FF, _D), lambda b, bex_ref: (bex_ref[b], 0, 0)),
        ],
        out_specs=pl.BlockSpec((_BLK, _D), lambda b, bex_ref: (b, 0)),
    )
    y = pl.pallas_call(
        _y_kernel,
        grid_spec=grid_spec2,
        out_shape=jax.ShapeDtypeStruct((_NSLOTS, _D), jnp.float32),
    )(bex, h, W2)

    yg = _gather_sc(y, pos_flat)

    out = pl.pallas_call(
        _combine_kernel,
        grid=(_T // 512,),
        in_specs=[
            pl.BlockSpec((512, 128), lambda i: (i, 0)),
            pl.BlockSpec((512, _D), lambda i: (i, 0)),
            pl.BlockSpec((512, _D), lambda i: (_T // 512 + i, 0)),
        ],
        out_specs=pl.BlockSpec((512, _D), lambda i: (i, 0)),
        out_shape=jax.ShapeDtypeStruct((_T, _D), jnp.float32),
    )(w128, yg, yg)
    return out


# speedup vs baseline: 1.0960x; 1.0960x over previous
import functools
import jax
import jax.numpy as jnp
from jax import lax
from jax.experimental import pallas as pl
from jax.experimental.pallas import tpu as pltpu
from jax.experimental.pallas import tpu_sc as plsc

_E = 8
_T = 2048
_D = 2048
_FF = 2048
_P = 2 * _T
_BLK = 256
_NSLOTS = _P + _E * _BLK
_NBLK = _NSLOTS // _BLK
_CH = 16
_ROUTE_CHUNK = 256


def _route_kernel(x_ref, wg_ref, pos_ref, w_ref, bex_ref, oh_ref, rank_ref):
    logits = jnp.dot(x_ref[...], wg_ref[...], preferred_element_type=jnp.float32)
    cols = lax.broadcasted_iota(jnp.int32, logits.shape, 1)
    v1 = jnp.max(logits, axis=1, keepdims=True)
    a1 = jnp.argmax(logits, axis=1).astype(jnp.int32)
    masked = jnp.where(cols == a1[:, None], -jnp.inf, logits)
    v2 = jnp.max(masked, axis=1, keepdims=True)
    a2 = jnp.argmax(masked, axis=1).astype(jnp.int32)
    r = jnp.exp(v2 - v1)
    w1c = 1.0 / (1.0 + r)
    w2c = r * w1c

    e_pair = jnp.concatenate([a1, a2], axis=0)
    oh = (e_pair[:, None] == lax.broadcasted_iota(jnp.int32, (_P, _E), 1))
    oh_ref[...] = oh.astype(jnp.float32)

    n_ch = _P // _ROUTE_CHUNK
    ri = lax.broadcasted_iota(jnp.int32, (_ROUTE_CHUNK, _ROUTE_CHUNK), 0)
    ci = lax.broadcasted_iota(jnp.int32, (_ROUTE_CHUNK, _ROUTE_CHUNK), 1)
    tril = (ci <= ri).astype(jnp.float32)

    def body(c, carry):
        rows = oh_ref[pl.ds(c * _ROUTE_CHUNK, _ROUTE_CHUNK), :]
        part = jnp.dot(tril, rows, preferred_element_type=jnp.float32) + carry
        rank_ref[pl.ds(c * _ROUTE_CHUNK, _ROUTE_CHUNK), :] = part
        return carry + jnp.sum(rows, axis=0, keepdims=True)

    total = lax.fori_loop(0, n_ch, body, jnp.zeros((1, _E), jnp.float32))

    padded = jnp.floor((total + float(_BLK - 1)) / float(_BLK)) * float(_BLK)
    ei = lax.broadcasted_iota(jnp.int32, (_E, _E), 0)
    ej = lax.broadcasted_iota(jnp.int32, (_E, _E), 1)
    sut = (ei < ej).astype(jnp.float32)
    base = jnp.dot(padded, sut, preferred_element_type=jnp.float32)

    rank = rank_ref[...]
    posm = base + rank - 1.0
    pos_pair = jnp.sum(jnp.where(oh, posm, 0.0), axis=1)
    pos0 = pos_pair[: _T].astype(jnp.int32)
    pos1 = pos_pair[_T:].astype(jnp.int32)

    lane = lax.broadcasted_iota(jnp.int32, (_T, 128), 1)
    pos_ref[...] = jnp.where(lane == 0, pos0[:, None],
                             jnp.where(lane == 1, pos1[:, None], 0))
    w_ref[...] = jnp.where(lane == 0, w1c,
                           jnp.where(lane == 1, w2c, 0.0))

    blane = lax.broadcasted_iota(jnp.int32, (8, 128), 1)
    s = (blane * _BLK).astype(jnp.float32)
    be = jnp.zeros((8, 128), jnp.float32)
    for e in range(_E):
        lane8 = lax.broadcasted_iota(jnp.int32, (1, _E), 1)
        b_e = jnp.sum(jnp.where(lane8 == e, base, 0.0))
        p_e = jnp.sum(jnp.where(lane8 == e, padded, 0.0))
        be = jnp.where((s >= b_e) & (s < b_e + p_e), float(e), be)
    nact = jnp.sum(padded) / float(_BLK)
    be = jnp.where((blane == _NBLK) & (lax.broadcasted_iota(jnp.int32, (8, 128), 0) == 0),
                   nact, be)
    bex_ref[...] = be.astype(jnp.int32)


def _h_kernel(bex_ref, xd_ref, w1_ref, h_ref):
    @pl.when(pl.program_id(0) < bex_ref[_NBLK])
    def _():
        h = jnp.dot(xd_ref[...], w1_ref[0], preferred_element_type=jnp.float32)
        h_ref[...] = (h * lax.logistic(h)).astype(jnp.bfloat16)


def _y_kernel(bex_ref, h_ref, w2_ref, y_ref):
    @pl.when(pl.program_id(0) < bex_ref[_NBLK])
    def _():
        y = jnp.dot(h_ref[...].astype(jnp.float32), w2_ref[0],
                    preferred_element_type=jnp.float32)
        u = pltpu.bitcast(y, jnp.uint32)
        lo = u[:, : _D // 2]
        hi = u[:, _D // 2 :]
        half = jnp.uint32(0x8000)
        topm = jnp.uint32(0xFFFF0000)
        packed = ((lo + half) >> 16) | ((hi + half) & topm)
        y_ref[...] = pltpu.bitcast(packed, jnp.int32)


def _combine_kernel(w_ref, y0_ref, y1_ref, out_ref):
    w = w_ref[...]
    lane = lax.broadcasted_iota(jnp.int32, w.shape, 1)
    w0 = jnp.sum(jnp.where(lane == 0, w, 0.0), axis=1, keepdims=True)
    w1 = jnp.sum(jnp.where(lane == 1, w, 0.0), axis=1, keepdims=True)
    u0 = pltpu.bitcast(y0_ref[...], jnp.uint32)
    u1 = pltpu.bitcast(y1_ref[...], jnp.uint32)
    topm = jnp.uint32(0xFFFF0000)
    lo0 = pltpu.bitcast(u0 << 16, jnp.float32)
    hi0 = pltpu.bitcast(u0 & topm, jnp.float32)
    lo1 = pltpu.bitcast(u1 << 16, jnp.float32)
    hi1 = pltpu.bitcast(u1 & topm, jnp.float32)
    out_ref[...] = jnp.concatenate(
        [w0 * lo0 + w1 * lo1, w0 * hi0 + w1 * hi1], axis=1)


@functools.lru_cache(maxsize=1)
def _make_sc_kernels():
    info = plsc.get_sparse_core_info()
    nw = info.num_cores * info.num_subcores
    per_w = _P // nw
    n_ch = per_w // _CH

    mesh = plsc.VectorSubcoreMesh(core_axis_name="c", subcore_axis_name="s")

    @functools.partial(
        pl.kernel, mesh=mesh,
        out_type=jax.ShapeDtypeStruct((_NSLOTS, _D), jnp.float32),
        scratch_types=[
            pltpu.VMEM((per_w,), jnp.int32),
            pltpu.VMEM((_CH, _D), jnp.float32),
            pltpu.VMEM((_CH, _D), jnp.float32),
            pltpu.SemaphoreType.DMA,
            pltpu.SemaphoreType.DMA,
        ],
    )
    def dispatch(x_hbm, pos_hbm, xd_hbm, pos_v, buf_a, buf_b, sem_a, sem_b):
        wid = lax.axis_index("s") * info.num_cores + lax.axis_index("c")
        p0 = wid * per_w
        row0 = lax.rem(p0, _T)
        pltpu.sync_copy(pos_hbm.at[pl.ds(p0, per_w)], pos_v)
        bufs = (buf_a, buf_b)
        sems = (sem_a, sem_b)
        descs = [None, None]
        for j in range(n_ch):
            b = j % 2
            if descs[b] is not None:
                descs[b].wait()
            pltpu.sync_copy(x_hbm.at[pl.ds(row0 + j * _CH, _CH)], bufs[b])
            idx = pos_v[pl.ds(j * _CH, _CH)]
            descs[b] = pltpu.async_copy(bufs[b], xd_hbm.at[idx], sems[b])
        for d in descs:
            if d is not None:
                d.wait()

    @functools.partial(
        pl.kernel, mesh=mesh,
        out_type=jax.ShapeDtypeStruct((_P, _D // 2), jnp.int32),
        scratch_types=[
            pltpu.VMEM((per_w,), jnp.int32),
            pltpu.VMEM((_CH, _D // 2), jnp.int32),
            pltpu.VMEM((_CH, _D // 2), jnp.int32),
            pltpu.SemaphoreType.DMA,
            pltpu.SemaphoreType.DMA,
        ],
    )
    def gather(y_hbm, pos_hbm, yg_hbm, pos_v, buf_a, buf_b, sem_a, sem_b):
        wid = lax.axis_index("s") * info.num_cores + lax.axis_index("c")
        p0 = wid * per_w
        pltpu.sync_copy(pos_hbm.at[pl.ds(p0, per_w)], pos_v)
        bufs = (buf_a, buf_b)
        sems = (sem_a, sem_b)
        descs = [None, None]
        for j in range(n_ch):
            b = j % 2
            if descs[b] is not None:
                descs[b].wait()
                pltpu.sync_copy(bufs[b], yg_hbm.at[pl.ds(p0 + (j - 2) * _CH, _CH)])
            idx = pos_v[pl.ds(j * _CH, _CH)]
            descs[b] = pltpu.async_copy(y_hbm.at[idx], bufs[b], sems[b])
        for j in range(n_ch - 2, n_ch):
            b = j % 2
            descs[b].wait()
            pltpu.sync_copy(bufs[b], yg_hbm.at[pl.ds(p0 + j * _CH, _CH)])

    return dispatch, gather


@jax.jit
def kernel(input, Wg, W1, W2):
    _dispatch_sc, _gather_sc = _make_sc_kernels()
    pos128, w128, bex128 = pl.pallas_call(
        _route_kernel,
        out_shape=[
            jax.ShapeDtypeStruct((_T, 128), jnp.int32),
            jax.ShapeDtypeStruct((_T, 128), jnp.float32),
            jax.ShapeDtypeStruct((8, 128), jnp.int32),
        ],
        scratch_shapes=[
            pltpu.VMEM((_P, _E), jnp.float32),
            pltpu.VMEM((_P, _E), jnp.float32),
        ],
    )(input, Wg)

    pos_flat = jnp.concatenate([pos128[:, 0], pos128[:, 1]], axis=0)
    bex = bex128[0, : _NBLK + 8]

    xd = _dispatch_sc(input, pos_flat)

    grid_spec = pltpu.PrefetchScalarGridSpec(
        num_scalar_prefetch=1,
        grid=(_NBLK,),
        in_specs=[
            pl.BlockSpec((_BLK, _D), lambda b, bex_ref: (b, 0)),
            pl.BlockSpec((1, _D, _FF), lambda b, bex_ref: (bex_ref[b], 0, 0)),
        ],
        out_specs=pl.BlockSpec((_BLK, _FF), lambda b, bex_ref: (b, 0)),
    )
    h = pl.pallas_call(
        _h_kernel,
        grid_spec=grid_spec,
        out_shape=jax.ShapeDtypeStruct((_NSLOTS, _FF), jnp.bfloat16),
    )(bex, xd, W1)

    grid_spec2 = pltpu.PrefetchScalarGridSpec(
        num_scalar_prefetch=1,
        grid=(_NBLK,),
        in_specs=[
            pl.BlockSpec((_BLK, _FF), lambda b, bex_ref: (b, 0)),
            pl.BlockSpec((1, _FF, _D), lambda b, bex_ref: (bex_ref[b], 0, 0)),
        ],
        out_specs=pl.BlockSpec((_BLK, _D // 2), lambda b, bex_ref: (b, 0)),
    )
    y = pl.pallas_call(
        _y_kernel,
        grid_spec=grid_spec2,
        out_shape=jax.ShapeDtypeStruct((_NSLOTS, _D // 2), jnp.int32),
    )(bex, h, W2)

    yg = _gather_sc(y, pos_flat)

    out = pl.pallas_call(
        _combine_kernel,
        grid=(_T // 512,),
        in_specs=[
            pl.BlockSpec((512, 128), lambda i: (i, 0)),
            pl.BlockSpec((512, _D // 2), lambda i: (i, 0)),
            pl.BlockSpec((512, _D // 2), lambda i: (_T // 512 + i, 0)),
        ],
        out_specs=pl.BlockSpec((512, _D), lambda i: (i, 0)),
        out_shape=jax.ShapeDtypeStruct((_T, _D), jnp.float32),
    )(w128, yg, yg)
    return out


# speedup vs baseline: 1.1234x; 1.0250x over previous
import functools
import jax
import jax.numpy as jnp
from jax import lax
from jax.experimental import pallas as pl
from jax.experimental.pallas import tpu as pltpu
from jax.experimental.pallas import tpu_sc as plsc

_E = 8
_T = 2048
_D = 2048
_FF = 2048
_P = 2 * _T
_BLK = 256
_NSLOTS = _P + _E * _BLK
_NBLK = _NSLOTS // _BLK
_CH = 16
_ROUTE_CHUNK = 256


def _route_kernel(x_ref, wg_ref, pos_ref, w_ref, bex_ref, xp_ref, oh_ref, rank_ref):
    xu = pltpu.bitcast(x_ref[...], jnp.uint32)
    xlo = xu[:, : _D // 2]
    xhi = xu[:, _D // 2 :]
    xhalf = jnp.uint32(0x8000)
    xtopm = jnp.uint32(0xFFFF0000)
    xp_ref[...] = pltpu.bitcast(
        ((xlo + xhalf) >> 16) | ((xhi + xhalf) & xtopm), jnp.int32)
    logits = jnp.dot(x_ref[...], wg_ref[...], preferred_element_type=jnp.float32)
    cols = lax.broadcasted_iota(jnp.int32, logits.shape, 1)
    v1 = jnp.max(logits, axis=1, keepdims=True)
    a1 = jnp.argmax(logits, axis=1).astype(jnp.int32)
    masked = jnp.where(cols == a1[:, None], -jnp.inf, logits)
    v2 = jnp.max(masked, axis=1, keepdims=True)
    a2 = jnp.argmax(masked, axis=1).astype(jnp.int32)
    r = jnp.exp(v2 - v1)
    w1c = 1.0 / (1.0 + r)
    w2c = r * w1c

    e_pair = jnp.concatenate([a1, a2], axis=0)
    oh = (e_pair[:, None] == lax.broadcasted_iota(jnp.int32, (_P, _E), 1))
    oh_ref[...] = oh.astype(jnp.float32)

    n_ch = _P // _ROUTE_CHUNK
    ri = lax.broadcasted_iota(jnp.int32, (_ROUTE_CHUNK, _ROUTE_CHUNK), 0)
    ci = lax.broadcasted_iota(jnp.int32, (_ROUTE_CHUNK, _ROUTE_CHUNK), 1)
    tril = (ci <= ri).astype(jnp.float32)

    def body(c, carry):
        rows = oh_ref[pl.ds(c * _ROUTE_CHUNK, _ROUTE_CHUNK), :]
        part = jnp.dot(tril, rows, preferred_element_type=jnp.float32) + carry
        rank_ref[pl.ds(c * _ROUTE_CHUNK, _ROUTE_CHUNK), :] = part
        return carry + jnp.sum(rows, axis=0, keepdims=True)

    total = lax.fori_loop(0, n_ch, body, jnp.zeros((1, _E), jnp.float32))

    padded = jnp.floor((total + float(_BLK - 1)) / float(_BLK)) * float(_BLK)
    ei = lax.broadcasted_iota(jnp.int32, (_E, _E), 0)
    ej = lax.broadcasted_iota(jnp.int32, (_E, _E), 1)
    sut = (ei < ej).astype(jnp.float32)
    base = jnp.dot(padded, sut, preferred_element_type=jnp.float32)

    rank = rank_ref[...]
    posm = base + rank - 1.0
    pos_pair = jnp.sum(jnp.where(oh, posm, 0.0), axis=1)
    pos0 = pos_pair[: _T].astype(jnp.int32)
    pos1 = pos_pair[_T:].astype(jnp.int32)

    lane = lax.broadcasted_iota(jnp.int32, (_T, 128), 1)
    pos_ref[...] = jnp.where(lane == 0, pos0[:, None],
                             jnp.where(lane == 1, pos1[:, None], 0))
    w_ref[...] = jnp.where(lane == 0, w1c,
                           jnp.where(lane == 1, w2c, 0.0))

    blane = lax.broadcasted_iota(jnp.int32, (8, 128), 1)
    s = (blane * _BLK).astype(jnp.float32)
    be = jnp.zeros((8, 128), jnp.float32)
    for e in range(_E):
        lane8 = lax.broadcasted_iota(jnp.int32, (1, _E), 1)
        b_e = jnp.sum(jnp.where(lane8 == e, base, 0.0))
        p_e = jnp.sum(jnp.where(lane8 == e, padded, 0.0))
        be = jnp.where((s >= b_e) & (s < b_e + p_e), float(e), be)
    nact = jnp.sum(padded) / float(_BLK)
    be = jnp.where((blane == _NBLK) & (lax.broadcasted_iota(jnp.int32, (8, 128), 0) == 0),
                   nact, be)
    bex_ref[...] = be.astype(jnp.int32)


def _h_kernel(bex_ref, xd_ref, w1_ref, h_ref):
    @pl.when(pl.program_id(0) < bex_ref[_NBLK])
    def _():
        u = pltpu.bitcast(xd_ref[...], jnp.uint32)
        topm = jnp.uint32(0xFFFF0000)
        x = jnp.concatenate(
            [pltpu.bitcast(u << 16, jnp.float32),
             pltpu.bitcast(u & topm, jnp.float32)], axis=1)
        h = jnp.dot(x, w1_ref[0], preferred_element_type=jnp.float32)
        h_ref[...] = (h * lax.logistic(h)).astype(jnp.bfloat16)


def _y_kernel(bex_ref, h_ref, w2_ref, y_ref):
    @pl.when(pl.program_id(0) < bex_ref[_NBLK])
    def _():
        y = jnp.dot(h_ref[...].astype(jnp.float32), w2_ref[0],
                    preferred_element_type=jnp.float32)
        u = pltpu.bitcast(y, jnp.uint32)
        lo = u[:, : _D // 2]
        hi = u[:, _D // 2 :]
        half = jnp.uint32(0x8000)
        topm = jnp.uint32(0xFFFF0000)
        packed = ((lo + half) >> 16) | ((hi + half) & topm)
        y_ref[...] = pltpu.bitcast(packed, jnp.int32)


def _combine_kernel(w_ref, y0_ref, y1_ref, out_ref):
    w = w_ref[...]
    lane = lax.broadcasted_iota(jnp.int32, w.shape, 1)
    w0 = jnp.sum(jnp.where(lane == 0, w, 0.0), axis=1, keepdims=True)
    w1 = jnp.sum(jnp.where(lane == 1, w, 0.0), axis=1, keepdims=True)
    u0 = pltpu.bitcast(y0_ref[...], jnp.uint32)
    u1 = pltpu.bitcast(y1_ref[...], jnp.uint32)
    topm = jnp.uint32(0xFFFF0000)
    lo0 = pltpu.bitcast(u0 << 16, jnp.float32)
    hi0 = pltpu.bitcast(u0 & topm, jnp.float32)
    lo1 = pltpu.bitcast(u1 << 16, jnp.float32)
    hi1 = pltpu.bitcast(u1 & topm, jnp.float32)
    out_ref[...] = jnp.concatenate(
        [w0 * lo0 + w1 * lo1, w0 * hi0 + w1 * hi1], axis=1)


@functools.lru_cache(maxsize=1)
def _make_sc_kernels():
    info = plsc.get_sparse_core_info()
    nw = info.num_cores * info.num_subcores
    per_w = _P // nw
    n_ch = per_w // _CH

    mesh = plsc.VectorSubcoreMesh(core_axis_name="c", subcore_axis_name="s")

    @functools.partial(
        pl.kernel, mesh=mesh,
        out_type=jax.ShapeDtypeStruct((_NSLOTS, _D // 2), jnp.int32),
        scratch_types=[
            pltpu.VMEM((per_w,), jnp.int32),
            pltpu.VMEM((_CH, _D // 2), jnp.int32),
            pltpu.VMEM((_CH, _D // 2), jnp.int32),
            pltpu.SemaphoreType.DMA,
            pltpu.SemaphoreType.DMA,
        ],
    )
    def dispatch(x_hbm, pos_hbm, xd_hbm, pos_v, buf_a, buf_b, sem_a, sem_b):
        wid = lax.axis_index("s") * info.num_cores + lax.axis_index("c")
        p0 = wid * per_w
        row0 = lax.rem(p0, _T)
        pltpu.sync_copy(pos_hbm.at[pl.ds(p0, per_w)], pos_v)
        bufs = (buf_a, buf_b)
        sems = (sem_a, sem_b)
        descs = [None, None]
        for j in range(n_ch):
            b = j % 2
            if descs[b] is not None:
                descs[b].wait()
            pltpu.sync_copy(x_hbm.at[pl.ds(row0 + j * _CH, _CH)], bufs[b])
            idx = pos_v[pl.ds(j * _CH, _CH)]
            descs[b] = pltpu.async_copy(bufs[b], xd_hbm.at[idx], sems[b])
        for d in descs:
            if d is not None:
                d.wait()

    @functools.partial(
        pl.kernel, mesh=mesh,
        out_type=jax.ShapeDtypeStruct((_P, _D // 2), jnp.int32),
        scratch_types=[
            pltpu.VMEM((per_w,), jnp.int32),
            pltpu.VMEM((_CH, _D // 2), jnp.int32),
            pltpu.VMEM((_CH, _D // 2), jnp.int32),
            pltpu.SemaphoreType.DMA,
            pltpu.SemaphoreType.DMA,
        ],
    )
    def gather(y_hbm, pos_hbm, yg_hbm, pos_v, buf_a, buf_b, sem_a, sem_b):
        wid = lax.axis_index("s") * info.num_cores + lax.axis_index("c")
        p0 = wid * per_w
        pltpu.sync_copy(pos_hbm.at[pl.ds(p0, per_w)], pos_v)
        bufs = (buf_a, buf_b)
        sems = (sem_a, sem_b)
        descs = [None, None]
        for j in range(n_ch):
            b = j % 2
            if descs[b] is not None:
                descs[b].wait()
                pltpu.sync_copy(bufs[b], yg_hbm.at[pl.ds(p0 + (j - 2) * _CH, _CH)])
            idx = pos_v[pl.ds(j * _CH, _CH)]
            descs[b] = pltpu.async_copy(y_hbm.at[idx], bufs[b], sems[b])
        for j in range(n_ch - 2, n_ch):
            b = j % 2
            descs[b].wait()
            pltpu.sync_copy(bufs[b], yg_hbm.at[pl.ds(p0 + j * _CH, _CH)])

    return dispatch, gather


@jax.jit
def kernel(input, Wg, W1, W2):
    _dispatch_sc, _gather_sc = _make_sc_kernels()
    pos128, w128, bex128, xp = pl.pallas_call(
        _route_kernel,
        out_shape=[
            jax.ShapeDtypeStruct((_T, 128), jnp.int32),
            jax.ShapeDtypeStruct((_T, 128), jnp.float32),
            jax.ShapeDtypeStruct((8, 128), jnp.int32),
            jax.ShapeDtypeStruct((_T, _D // 2), jnp.int32),
        ],
        scratch_shapes=[
            pltpu.VMEM((_P, _E), jnp.float32),
            pltpu.VMEM((_P, _E), jnp.float32),
        ],
    )(input, Wg)

    pos_flat = jnp.concatenate([pos128[:, 0], pos128[:, 1]], axis=0)
    bex = bex128[0, : _NBLK + 8]

    xd = _dispatch_sc(xp, pos_flat)

    grid_spec = pltpu.PrefetchScalarGridSpec(
        num_scalar_prefetch=1,
        grid=(_NBLK,),
        in_specs=[
            pl.BlockSpec((_BLK, _D // 2), lambda b, bex_ref: (b, 0)),
            pl.BlockSpec((1, _D, _FF), lambda b, bex_ref: (bex_ref[b], 0, 0)),
        ],
        out_specs=pl.BlockSpec((_BLK, _FF), lambda b, bex_ref: (b, 0)),
    )
    h = pl.pallas_call(
        _h_kernel,
        grid_spec=grid_spec,
        out_shape=jax.ShapeDtypeStruct((_NSLOTS, _FF), jnp.bfloat16),
    )(bex, xd, W1)

    grid_spec2 = pltpu.PrefetchScalarGridSpec(
        num_scalar_prefetch=1,
        grid=(_NBLK,),
        in_specs=[
            pl.BlockSpec((_BLK, _FF), lambda b, bex_ref: (b, 0)),
            pl.BlockSpec((1, _FF, _D), lambda b, bex_ref: (bex_ref[b], 0, 0)),
        ],
        out_specs=pl.BlockSpec((_BLK, _D // 2), lambda b, bex_ref: (b, 0)),
    )
    y = pl.pallas_call(
        _y_kernel,
        grid_spec=grid_spec2,
        out_shape=jax.ShapeDtypeStruct((_NSLOTS, _D // 2), jnp.int32),
    )(bex, h, W2)

    yg = _gather_sc(y, pos_flat)

    out = pl.pallas_call(
        _combine_kernel,
        grid=(_T // 512,),
        in_specs=[
            pl.BlockSpec((512, 128), lambda i: (i, 0)),
            pl.BlockSpec((512, _D // 2), lambda i: (i, 0)),
            pl.BlockSpec((512, _D // 2), lambda i: (_T // 512 + i, 0)),
        ],
        out_specs=pl.BlockSpec((512, _D), lambda i: (i, 0)),
        out_shape=jax.ShapeDtypeStruct((_T, _D), jnp.float32),
    )(w128, yg, yg)
    return out
